# Initial kernel scaffold; baseline (speedup 1.0000x reference)
import jax
import jax.numpy as jnp
from jax.experimental import pallas as pl


def kernel(x, z, idx_pa):
    raise NotImplementedError("write your pallas kernel here")


# speedup vs baseline: 8.9574x; 8.9574x over previous
import functools

import jax
import jax.numpy as jnp
from jax import lax
from jax.experimental import pallas as pl
from jax.experimental.pallas import tpu as pltpu
from jax.experimental.pallas import tpu_sc as plsc

B = 32
S = 8192
NQ = 16384
DX = 64
DZ = 32

IW = 128
KC = 4
CHUNK = KC * IW
NCHUNK = NQ // CHUNK

_MESH = plsc.VectorSubcoreMesh(core_axis_name="c", subcore_axis_name="s")


def _gather_body(xf, zf, idxf, xg, zg, idx_v, xbuf, zbuf, sem):
    nc = 2
    wid = lax.axis_index("s") * nc + lax.axis_index("c")
    base = wid * S
    qbase = wid * NQ
    irow0 = wid * (NQ // IW)

    def chunk(c, _):
        pltpu.sync_copy(idxf.at[pl.ds(irow0 + c * KC, KC)], idx_v)
        for r in range(KC):
            for g in range(IW // 16):
                idx_v[r, pl.ds(g * 16, 16)] = idx_v[r, pl.ds(g * 16, 16)] + base
        cps = []
        for r in range(KC):
            cps.append(pltpu.async_copy(
                xf.at[idx_v.at[r]], xbuf.at[pl.ds(r * IW, IW)], sem))
            cps.append(pltpu.async_copy(
                zf.at[idx_v.at[r]], zbuf.at[pl.ds(r * IW, IW)], sem))
        for cp in cps:
            cp.wait()
        out0 = qbase + c * CHUNK
        pltpu.sync_copy(xbuf, xg.at[pl.ds(out0, CHUNK)])
        pltpu.sync_copy(zbuf, zg.at[pl.ds(out0, CHUNK)])
        return ()

    lax.fori_loop(0, NCHUNK, chunk, (), unroll=False)


@functools.partial(jax.jit, donate_argnums=())
def _run(xf, zf, idxf):
    return pl.kernel(
        _gather_body,
        out_type=(
            jax.ShapeDtypeStruct((B * NQ, DX), jnp.float32),
            jax.ShapeDtypeStruct((B * NQ, DZ), jnp.float32),
        ),
        mesh=_MESH,
        scratch_types=[
            pltpu.VMEM((KC, IW), jnp.int32),
            pltpu.VMEM((CHUNK, DX), jnp.float32),
            pltpu.VMEM((CHUNK, DZ), jnp.float32),
            pltpu.SemaphoreType.DMA,
        ],
        compiler_params=pltpu.CompilerParams(use_tc_tiling_on_sc=False),
    )(xf, zf, idxf)


def kernel(x, z, idx_pa):
    xf = x.reshape(B * S, DX)
    zf = z.reshape(B * S, DZ)
    idxf = idx_pa.astype(jnp.int32).reshape(B * NQ // IW, IW)
    xg, zg = _run(xf, zf, idxf)
    return xg.reshape(B, NQ, DX), zg.reshape(B, NQ, DZ)


# speedup vs baseline: 9.2652x; 1.0344x over previous
import functools

import jax
import jax.numpy as jnp
from jax import lax
from jax.experimental import pallas as pl
from jax.experimental.pallas import tpu as pltpu
from jax.experimental.pallas import tpu_sc as plsc

B = 32
S = 8192
NQ = 16384
DX = 64
DZ = 32

IW = 128
KC = 4
CHUNK = KC * IW
NCHUNK = NQ // CHUNK

_MESH = plsc.VectorSubcoreMesh(core_axis_name="c", subcore_axis_name="s")


def _gather_body(xf, zf, idxf, xg, zg,
                 idx_v, xbuf, zbuf, isem0, isem1, gsem, wsem0, wsem1):
    nc = 2
    wid = lax.axis_index("s") * nc + lax.axis_index("c")
    base = wid * S
    qbase = wid * NQ
    irow0 = wid * (NQ // IW)
    isems = (isem0, isem1)
    wsems = (wsem0, wsem1)

    def idx_copy(c, b, sem):
        return pltpu.make_async_copy(
            idxf.at[pl.ds(irow0 + c * KC, KC)], idx_v.at[b], sem)

    def out_copies(c, b, sem):
        out0 = qbase + c * CHUNK
        return (pltpu.make_async_copy(xbuf.at[b], xg.at[pl.ds(out0, CHUNK)], sem),
                pltpu.make_async_copy(zbuf.at[b], zg.at[pl.ds(out0, CHUNK)], sem))

    idx_copy(0, 0, isems[0]).start()

    @pl.loop(0, NCHUNK, step=2)
    def _(c2):
        for b in range(2):
            c = c2 + b
            idx_copy(c, b, isems[b]).wait()
            for r in range(KC):
                for g in range(IW // 16):
                    idx_v[b, r, pl.ds(g * 16, 16)] = (
                        idx_v[b, r, pl.ds(g * 16, 16)] + base)
            @pl.when(c + 1 < NCHUNK)
            def _():
                idx_copy(c + 1, 1 - b, isems[1 - b]).start()
            @pl.when(c >= 2)
            def _():
                for cp in out_copies(c - 2, b, wsems[b]):
                    cp.wait()
            cps = []
            for r in range(KC):
                cps.append(pltpu.async_copy(
                    xf.at[idx_v.at[b, r]], xbuf.at[b, pl.ds(r * IW, IW)], gsem))
                cps.append(pltpu.async_copy(
                    zf.at[idx_v.at[b, r]], zbuf.at[b, pl.ds(r * IW, IW)], gsem))
            for cp in cps:
                cp.wait()
            for cp in out_copies(c, b, wsems[b]):
                cp.start()

    for b in range(2):
        for cp in out_copies(NCHUNK - 2 + b, b, wsems[b]):
            cp.wait()


@jax.jit
def _run(xf, zf, idxf):
    return pl.kernel(
        _gather_body,
        out_type=(
            jax.ShapeDtypeStruct((B * NQ, DX), jnp.float32),
            jax.ShapeDtypeStruct((B * NQ, DZ), jnp.float32),
        ),
        mesh=_MESH,
        scratch_types=[
            pltpu.VMEM((2, KC, IW), jnp.int32),
            pltpu.VMEM((2, CHUNK, DX), jnp.float32),
            pltpu.VMEM((2, CHUNK, DZ), jnp.float32),
            pltpu.SemaphoreType.DMA,
            pltpu.SemaphoreType.DMA,
            pltpu.SemaphoreType.DMA,
            pltpu.SemaphoreType.DMA,
            pltpu.SemaphoreType.DMA,
        ],
        compiler_params=pltpu.CompilerParams(use_tc_tiling_on_sc=False),
    )(xf, zf, idxf)


def kernel(x, z, idx_pa):
    xf = x.reshape(B * S, DX)
    zf = z.reshape(B * S, DZ)
    idxf = idx_pa.astype(jnp.int32).reshape(B * NQ // IW, IW)
    xg, zg = _run(xf, zf, idxf)
    return xg.reshape(B, NQ, DX), zg.reshape(B, NQ, DZ)


# speedup vs baseline: 14.4033x; 1.5546x over previous
import functools

import jax
import jax.numpy as jnp
from jax import lax
from jax.experimental import pallas as pl
from jax.experimental.pallas import tpu as pltpu
from jax.experimental.pallas import tpu_sc as plsc

B = 32
S = 8192
NQ = 16384
DX = 64
DZ = 32
NV = S // 128
NVQ = NQ // 128
NGRP = NQ // 16
OCROWS = 64
NOC = NQ // (8 * 128)

_MESH = plsc.VectorSubcoreMesh(core_axis_name="c", subcore_axis_name="s")


def _gather_body(xq, zq, idxq, xgq, zgq,
                 idxbuf, rowbuf, lanebuf, chunk, outbuf, wsem0, wsem1):
    nc = 2
    b = lax.axis_index("s") * nc + lax.axis_index("c")
    tb = b // 8
    rb = b % 8
    wsems = (wsem0, wsem1)

    pltpu.sync_copy(idxq.at[tb, :, pl.ds(rb, 1), :], idxbuf)

    @pl.loop(0, NVQ)
    def _(vq):
        for g in range(8):
            grp = vq * 8 + g
            s = idxbuf[vq, 0, pl.ds(g * 16, 16)]
            rowbuf[grp, :] = (s >> 7) << 3
            lanebuf[grp, :] = s & 127

    def run_job(src, dst, job, first):
        pltpu.sync_copy(src.at[pl.ds(job * 512, 512)], chunk)
        obase = job * (NOC * OCROWS)

        @pl.loop(0, NOC, step=2)
        def _(oc2):
            for p in range(2):
                oc = oc2 + p
                cond = (oc >= 2) if first else (oc >= 0)

                @pl.when(cond)
                def _():
                    pltpu.make_async_copy(
                        outbuf.at[p],
                        dst.at[pl.ds(obase, OCROWS)],
                        wsems[p]).wait()

                @pl.loop(0, 64)
                def _(ig):
                    grp = oc * 64 + ig
                    rv = rowbuf[grp, :]
                    lv = lanebuf[grp, :]
                    lrow = (ig >> 3) * 8
                    lg = ig & 7
                    for r in range(8):
                        vals = plsc.load_gather(chunk, [rv + r, lv])
                        outbuf[p, lrow + r, pl.ds(lg * 16, 16)] = vals

                pltpu.async_copy(
                    outbuf.at[p],
                    dst.at[pl.ds(obase + oc * OCROWS, OCROWS)],
                    wsems[p])

    for u in range(8):
        run_job(xq, xgq, b * 8 + u, first=(u == 0))
    for w in range(4):
        run_job(zq, zgq, b * 4 + w, first=False)

    for p in range(2):
        pltpu.make_async_copy(
            outbuf.at[p],
            zgq.at[pl.ds(0, OCROWS)],
            wsems[p]).wait()


@jax.jit
def _run(xq, zq, idxq):
    return pl.kernel(
        _gather_body,
        out_type=(
            jax.ShapeDtypeStruct((B * 8 * NVQ * 8, 128), jnp.float32),
            jax.ShapeDtypeStruct((B * 4 * NVQ * 8, 128), jnp.float32),
        ),
        mesh=_MESH,
        scratch_types=[
            pltpu.VMEM((NVQ, 1, 128), jnp.int32),
            pltpu.VMEM((NGRP, 16), jnp.int32),
            pltpu.VMEM((NGRP, 16), jnp.int32),
            pltpu.VMEM((512, 128), jnp.float32),
            pltpu.VMEM((2, OCROWS, 128), jnp.float32),
            pltpu.SemaphoreType.DMA,
            pltpu.SemaphoreType.DMA,
        ],
        compiler_params=pltpu.CompilerParams(
            use_tc_tiling_on_sc=False, needs_layout_passes=False),
    )(xq, zq, idxq)


def kernel(x, z, idx_pa):
    xq = (x.transpose(0, 2, 1)
           .reshape(B, 8, 8, NV, 128)
           .transpose(0, 1, 3, 2, 4)
           .reshape(B * 8 * NV * 8, 128))
    zq = (z.transpose(0, 2, 1)
           .reshape(B, 4, 8, NV, 128)
           .transpose(0, 1, 3, 2, 4)
           .reshape(B * 4 * NV * 8, 128))
    idxq = (idx_pa.astype(jnp.int32)
            .reshape(4, 8, NVQ, 128)
            .transpose(0, 2, 1, 3))
    xgq, zgq = _run(xq, zq, idxq)
    xg = (xgq.reshape(B, 8, NVQ, 8, 128)
             .transpose(0, 1, 3, 2, 4)
             .reshape(B, DX, NQ)
             .transpose(0, 2, 1))
    zg = (zgq.reshape(B, 4, NVQ, 8, 128)
             .transpose(0, 1, 3, 2, 4)
             .reshape(B, DZ, NQ)
             .transpose(0, 2, 1))
    return xg, zg


# speedup vs baseline: 47.7129x; 3.3126x over previous
import functools

import jax
import jax.numpy as jnp
from jax import lax
from jax.experimental import pallas as pl
from jax.experimental.pallas import tpu as pltpu
from jax.experimental.pallas import tpu_sc as plsc

B = 32
S = 8192
NQ = 16384
DX = 64
DZ = 32
NV = S // 128
NVQ = NQ // 128
NGRP = NQ // 16
OCROWS = 64
NOC = NQ // (8 * 128)

_MESH = plsc.VectorSubcoreMesh(core_axis_name="c", subcore_axis_name="s")


def _gather_body(xq, zq, idxq, xgq, zgq,
                 idxbuf, rowbuf, lanebuf, chunk, outbuf, wsem0, wsem1):
    nc = 2
    b = lax.axis_index("s") * nc + lax.axis_index("c")
    tb = b // 8
    rb = b % 8
    wsems = (wsem0, wsem1)

    pltpu.sync_copy(idxq.at[tb, :, pl.ds(rb, 1), :], idxbuf)

    @pl.loop(0, NVQ)
    def _(vq):
        for g in range(8):
            grp = vq * 8 + g
            s = idxbuf[vq, 0, pl.ds(g * 16, 16)]
            rowbuf[grp, :] = (s >> 7) << 3
            lanebuf[grp, :] = s & 127

    def run_job(src, dst, job, first):
        pltpu.sync_copy(src.at[pl.ds(job * 512, 512)], chunk)
        obase = job * (NOC * OCROWS)

        @pl.loop(0, NOC, step=2)
        def _(oc2):
            for p in range(2):
                oc = oc2 + p
                cond = (oc >= 2) if first else (oc >= 0)

                @pl.when(cond)
                def _():
                    pltpu.make_async_copy(
                        outbuf.at[p],
                        dst.at[pl.ds(obase, OCROWS)],
                        wsems[p]).wait()

                @plsc.parallel_loop(0, 64, unroll=4)
                def _(ig):
                    grp = oc * 64 + ig
                    rv = rowbuf[grp, :]
                    lv = lanebuf[grp, :]
                    lrow = (ig >> 3) * 8
                    lg = ig & 7
                    for r in range(8):
                        vals = plsc.load_gather(chunk, [rv + r, lv])
                        outbuf[p, lrow + r, pl.ds(lg * 16, 16)] = vals

                pltpu.async_copy(
                    outbuf.at[p],
                    dst.at[pl.ds(obase + oc * OCROWS, OCROWS)],
                    wsems[p])

    for u in range(8):
        run_job(xq, xgq, b * 8 + u, first=(u == 0))
    for w in range(4):
        run_job(zq, zgq, b * 4 + w, first=False)

    for p in range(2):
        pltpu.make_async_copy(
            outbuf.at[p],
            zgq.at[pl.ds(0, OCROWS)],
            wsems[p]).wait()


@jax.jit
def _run(xq, zq, idxq):
    return pl.kernel(
        _gather_body,
        out_type=(
            jax.ShapeDtypeStruct((B * 8 * NVQ * 8, 128), jnp.float32),
            jax.ShapeDtypeStruct((B * 4 * NVQ * 8, 128), jnp.float32),
        ),
        mesh=_MESH,
        scratch_types=[
            pltpu.VMEM((NVQ, 1, 128), jnp.int32),
            pltpu.VMEM((NGRP, 16), jnp.int32),
            pltpu.VMEM((NGRP, 16), jnp.int32),
            pltpu.VMEM((512, 128), jnp.float32),
            pltpu.VMEM((2, OCROWS, 128), jnp.float32),
            pltpu.SemaphoreType.DMA,
            pltpu.SemaphoreType.DMA,
        ],
        compiler_params=pltpu.CompilerParams(
            use_tc_tiling_on_sc=False, needs_layout_passes=False),
    )(xq, zq, idxq)


def kernel(x, z, idx_pa):
    xq = (x.transpose(0, 2, 1)
           .reshape(B, 8, 8, NV, 128)
           .transpose(0, 1, 3, 2, 4)
           .reshape(B * 8 * NV * 8, 128))
    zq = (z.transpose(0, 2, 1)
           .reshape(B, 4, 8, NV, 128)
           .transpose(0, 1, 3, 2, 4)
           .reshape(B * 4 * NV * 8, 128))
    idxq = (idx_pa.astype(jnp.int32)
            .reshape(4, 8, NVQ, 128)
            .transpose(0, 2, 1, 3))
    xgq, zgq = _run(xq, zq, idxq)
    xg = (xgq.reshape(B, 8, NVQ, 8, 128)
             .transpose(0, 1, 3, 2, 4)
             .reshape(B, DX, NQ)
             .transpose(0, 2, 1))
    zg = (zgq.reshape(B, 4, NVQ, 8, 128)
             .transpose(0, 1, 3, 2, 4)
             .reshape(B, DZ, NQ)
             .transpose(0, 2, 1))
    return xg, zg


# speedup vs baseline: 58.1873x; 1.2195x over previous
import functools

import jax
import jax.numpy as jnp
from jax import lax
from jax.experimental import pallas as pl
from jax.experimental.pallas import tpu as pltpu
from jax.experimental.pallas import tpu_sc as plsc

B = 32
S = 8192
NQ = 16384
DX = 64
DZ = 32
NV = S // 128
NVQ = NQ // 128
OCROWS = 128
NOC = NQ * 8 // (OCROWS * 128)
GPO = OCROWS // 8 * 16 // 16

_MESH = plsc.VectorSubcoreMesh(core_axis_name="c", subcore_axis_name="s")


def _gather_body(xq, zq, idxq, xgq, zgq, idxbuf, chunk, outbuf, wsem0, wsem1):
    nc = 2
    b = lax.axis_index("s") * nc + lax.axis_index("c")
    tb = b // 8
    rb = b % 8
    wsems = (wsem0, wsem1)

    pltpu.sync_copy(idxq.at[tb, :, pl.ds(rb, 1), :], idxbuf)

    @plsc.parallel_loop(0, NVQ, unroll=2)
    def _(vq):
        for g in range(8):
            s = idxbuf[vq, 0, pl.ds(g * 16, 16)]
            idxbuf[vq, 0, pl.ds(g * 16, 16)] = ((s >> 7) << 10) | (s & 127)

    def run_job(src, dst, job, first):
        pltpu.sync_copy(src.at[pl.ds(job * 512, 512)], chunk)
        obase = job * (NOC * OCROWS)

        @pl.loop(0, NOC, step=2)
        def _(oc2):
            for p in range(2):
                oc = oc2 + p
                cond = (oc >= 2) if first else (oc >= 0)

                @pl.when(cond)
                def _():
                    pltpu.make_async_copy(
                        outbuf.at[p],
                        dst.at[pl.ds(obase, OCROWS)],
                        wsems[p]).wait()

                @plsc.parallel_loop(0, GPO, unroll=4)
                def _(ig):
                    grp = oc * GPO + ig
                    vq = grp >> 3
                    g = grp & 7
                    pk = idxbuf[vq, 0, pl.ds(g * 16, 16)]
                    rv = pk >> 7
                    lv = pk & 127
                    lrow = (ig >> 3) * 8
                    lg = ig & 7
                    for r in range(8):
                        vals = plsc.load_gather(chunk, [rv + r, lv])
                        outbuf[p, lrow + r, pl.ds(lg * 16, 16)] = vals

                pltpu.async_copy(
                    outbuf.at[p],
                    dst.at[pl.ds(obase + oc * OCROWS, OCROWS)],
                    wsems[p])

    for u in range(8):
        run_job(xq, xgq, b * 8 + u, first=(u == 0))
    for w in range(4):
        run_job(zq, zgq, b * 4 + w, first=False)

    for p in range(2):
        pltpu.make_async_copy(
            outbuf.at[p],
            zgq.at[pl.ds(0, OCROWS)],
            wsems[p]).wait()


@jax.jit
def _run(xq, zq, idxq):
    return pl.kernel(
        _gather_body,
        out_type=(
            jax.ShapeDtypeStruct((B * 8 * NVQ * 8, 128), jnp.float32),
            jax.ShapeDtypeStruct((B * 4 * NVQ * 8, 128), jnp.float32),
        ),
        mesh=_MESH,
        scratch_types=[
            pltpu.VMEM((NVQ, 1, 128), jnp.int32),
            pltpu.VMEM((512, 128), jnp.float32),
            pltpu.VMEM((2, OCROWS, 128), jnp.float32),
            pltpu.SemaphoreType.DMA,
            pltpu.SemaphoreType.DMA,
        ],
        compiler_params=pltpu.CompilerParams(
            use_tc_tiling_on_sc=False, needs_layout_passes=False),
    )(xq, zq, idxq)


def kernel(x, z, idx_pa):
    xq = (x.transpose(0, 2, 1)
           .reshape(B, 8, 8, NV, 128)
           .transpose(0, 1, 3, 2, 4)
           .reshape(B * 8 * NV * 8, 128))
    zq = (z.transpose(0, 2, 1)
           .reshape(B, 4, 8, NV, 128)
           .transpose(0, 1, 3, 2, 4)
           .reshape(B * 4 * NV * 8, 128))
    idxq = (idx_pa.astype(jnp.int32)
            .reshape(4, 8, NVQ, 128)
            .transpose(0, 2, 1, 3))
    xgq, zgq = _run(xq, zq, idxq)
    xg = (xgq.reshape(B, 8, NVQ, 8, 128)
             .transpose(0, 1, 3, 2, 4)
             .reshape(B, DX, NQ)
             .transpose(0, 2, 1))
    zg = (zgq.reshape(B, 4, NVQ, 8, 128)
             .transpose(0, 1, 3, 2, 4)
             .reshape(B, DZ, NQ)
             .transpose(0, 2, 1))
    return xg, zg
